# Initial kernel scaffold; baseline (speedup 1.0000x reference)
import jax
import jax.numpy as jnp
from jax.experimental import pallas as pl


def kernel(x, edge_index, Wl0, bl0, Wr0, Wl1, bl1, Wr1, Wl2, bl2, Wr2, W_fc, b_fc):
    raise NotImplementedError("write your pallas kernel here")


# speedup vs baseline: 3.3635x; 3.3635x over previous
import functools

import jax
import jax.numpy as jnp
from jax import lax
from jax.experimental import pallas as pl
from jax.experimental.pallas import tpu as pltpu
from jax.experimental.pallas import tpu_sc as plsc

_N = 10000
_NPAD = 10240
_E = 320000
_F = 128
_H = 128
_OUT = 64
_NC = 2
_NS = 16
_CH = 80
_EPT = _E // (_NC * _NS)
_ROWS_PT = _NPAD // _NS
_MBLK = 128


def _seg_body(p, src, dst, zeros, out, src_v, dst_v, rows_v, acc, sem):
    c = lax.axis_index("c")
    s = lax.axis_index("s")

    pltpu.sync_copy(zeros, acc.at[pl.ds(s * _ROWS_PT, _ROWS_PT)])
    plsc.subcore_barrier()

    base = (c * _NS + s) * _EPT

    def _win(g, _):
        off = base + g * _CH
        pltpu.sync_copy(src.at[pl.ds(off, _CH)], src_v)
        pltpu.sync_copy(dst.at[pl.ds(off, _CH)], dst_v)
        pltpu.async_copy(p.at[src_v], rows_v, sem).wait()
        pltpu.sync_copy(rows_v, acc.at[dst_v], add=True)
        return 0

    lax.fori_loop(0, _EPT // _CH, _win, 0)
    plsc.subcore_barrier()
    pltpu.sync_copy(acc.at[pl.ds(s * _ROWS_PT, _ROWS_PT)],
                    out.at[c, pl.ds(s * _ROWS_PT, _ROWS_PT)])


def _cnt_body(dst, ones, zeros, out, dst_v, ones_v, acc):
    c = lax.axis_index("c")
    s = lax.axis_index("s")

    pltpu.sync_copy(ones, ones_v)
    pltpu.sync_copy(zeros, acc.at[pl.ds(s * _ROWS_PT, _ROWS_PT)])
    plsc.subcore_barrier()

    base = (c * _NS + s) * _EPT

    def _win(g, _):
        pltpu.sync_copy(dst.at[pl.ds(base + g * _CH, _CH)], dst_v)
        pltpu.sync_copy(ones_v, acc.at[dst_v], add=True)
        return 0

    lax.fori_loop(0, _EPT // _CH, _win, 0)
    plsc.subcore_barrier()
    pltpu.sync_copy(acc.at[pl.ds(s * _ROWS_PT, _ROWS_PT)],
                    out.at[c, pl.ds(s * _ROWS_PT, _ROWS_PT)])


@functools.cache
def _seg_call():
    mesh = plsc.VectorSubcoreMesh(core_axis_name="c", subcore_axis_name="s",
                                  num_cores=_NC, num_subcores=_NS)
    return pl.kernel(
        _seg_body,
        out_type=jax.ShapeDtypeStruct((_NC, _NPAD, _H), jnp.float32),
        mesh=mesh,
        scratch_types=[
            pltpu.VMEM((_CH,), jnp.int32),
            pltpu.VMEM((_CH,), jnp.int32),
            pltpu.VMEM((_CH, _H), jnp.float32),
            pltpu.VMEM_SHARED((_NPAD, _H), jnp.float32),
            pltpu.SemaphoreType.DMA,
        ],
    )


@functools.cache
def _cnt_call():
    mesh = plsc.VectorSubcoreMesh(core_axis_name="c", subcore_axis_name="s",
                                  num_cores=_NC, num_subcores=_NS)
    return pl.kernel(
        _cnt_body,
        out_type=jax.ShapeDtypeStruct((_NC, _NPAD, _H), jnp.float32),
        mesh=mesh,
        scratch_types=[
            pltpu.VMEM((_CH,), jnp.int32),
            pltpu.VMEM((_CH, _H), jnp.float32),
            pltpu.VMEM_SHARED((_NPAD, _H), jnp.float32),
        ],
    )


def _mm_p_kernel(h_ref, w_ref, o_ref):
    o_ref[...] = jnp.dot(h_ref[...], w_ref[...],
                         preferred_element_type=jnp.float32)


def _mm_p(h, wl):
    return pl.pallas_call(
        _mm_p_kernel,
        grid=(_NPAD // _MBLK,),
        in_specs=[
            pl.BlockSpec((_MBLK, _H), lambda i: (i, 0)),
            pl.BlockSpec((_H, _H), lambda i: (0, 0)),
        ],
        out_specs=pl.BlockSpec((_MBLK, _H), lambda i: (i, 0)),
        out_shape=jax.ShapeDtypeStruct((_NPAD, _H), jnp.float32),
    )(h, wl)


def _combine_kernel(s0, s1, c0, c1, h, wr, bl, o_ref):
    cnt = c0[:, 0:1] + c1[:, 0:1]
    inv = 1.0 / jnp.maximum(cnt, 1.0)
    mean = (s0[...] + s1[...]) * inv
    mm = jnp.dot(h[...], wr[...], preferred_element_type=jnp.float32)
    o_ref[...] = jnp.maximum(mean + mm + bl[...], 0.0)


def _combine(s0, s1, c0, c1, h, wr, bl):
    return pl.pallas_call(
        _combine_kernel,
        grid=(_NPAD // _MBLK,),
        in_specs=[
            pl.BlockSpec((_MBLK, _H), lambda i: (i, 0)),
            pl.BlockSpec((_MBLK, _H), lambda i: (i, 0)),
            pl.BlockSpec((_MBLK, _H), lambda i: (i, 0)),
            pl.BlockSpec((_MBLK, _H), lambda i: (i, 0)),
            pl.BlockSpec((_MBLK, _H), lambda i: (i, 0)),
            pl.BlockSpec((_H, _H), lambda i: (0, 0)),
            pl.BlockSpec((1, _H), lambda i: (0, 0)),
        ],
        out_specs=pl.BlockSpec((_MBLK, _H), lambda i: (i, 0)),
        out_shape=jax.ShapeDtypeStruct((_NPAD, _H), jnp.float32),
    )(s0, s1, c0, c1, h, wr, bl)


def _fc_kernel(h1, h2, h3, w1, w2, w3, b, o_ref):
    acc = jnp.dot(h1[...], w1[...], preferred_element_type=jnp.float32)
    acc += jnp.dot(h2[...], w2[...], preferred_element_type=jnp.float32)
    acc += jnp.dot(h3[...], w3[...], preferred_element_type=jnp.float32)
    o_ref[...] = acc + b[...]


def _fc(h1, h2, h3, w1, w2, w3, b):
    return pl.pallas_call(
        _fc_kernel,
        grid=(_NPAD // _MBLK,),
        in_specs=[
            pl.BlockSpec((_MBLK, _H), lambda i: (i, 0)),
            pl.BlockSpec((_MBLK, _H), lambda i: (i, 0)),
            pl.BlockSpec((_MBLK, _H), lambda i: (i, 0)),
            pl.BlockSpec((_H, _OUT), lambda i: (0, 0)),
            pl.BlockSpec((_H, _OUT), lambda i: (0, 0)),
            pl.BlockSpec((_H, _OUT), lambda i: (0, 0)),
            pl.BlockSpec((1, _OUT), lambda i: (0, 0)),
        ],
        out_specs=pl.BlockSpec((_MBLK, _OUT), lambda i: (i, 0)),
        out_shape=jax.ShapeDtypeStruct((_NPAD, _OUT), jnp.float32),
    )(h1, h2, h3, w1, w2, w3, b)


def kernel(x, edge_index, Wl0, bl0, Wr0, Wl1, bl1, Wr1, Wl2, bl2, Wr2,
           W_fc, b_fc):
    src = edge_index[0]
    dst = edge_index[1]

    xpad = jnp.pad(x, ((0, _NPAD - _N), (0, 0)))
    zeros_h = jnp.zeros((_ROWS_PT, _H), jnp.float32)
    ones_h = jnp.ones((_CH, _H), jnp.float32)

    cnt = _cnt_call()(dst, ones_h, zeros_h)

    h = xpad
    hs = []
    for Wl, bl, Wr in ((Wl0, bl0, Wr0), (Wl1, bl1, Wr1), (Wl2, bl2, Wr2)):
        p = _mm_p(h, Wl)
        ssum = _seg_call()(p, src, dst, zeros_h)
        h = _combine(ssum[0], ssum[1], cnt[0], cnt[1], h, Wr,
                     bl.reshape(1, _H))
        hs.append(h)

    out = _fc(hs[0], hs[1], hs[2], W_fc[0:_H], W_fc[_H:2 * _H],
              W_fc[2 * _H:3 * _H], b_fc.reshape(1, _OUT))
    return out[:_N]
